# Initial kernel scaffold; baseline (speedup 1.0000x reference)
import jax
import jax.numpy as jnp
from jax.experimental import pallas as pl


def kernel(x, embedding, new_embedding):
    raise NotImplementedError("write your pallas kernel here")


# speedup vs baseline: 5.3139x; 5.3139x over previous
import functools

import jax
import jax.numpy as jnp
from jax import lax
from jax.experimental import pallas as pl
from jax.experimental.pallas import tpu as pltpu
from jax.experimental.pallas import tpu_sc as plsc

NUM_EMB = 100000
NUM_NEW = 128
FEAT = 64
NC, NS, LANES = 2, 16, 16
NW = NC * NS
CHUNK = 512
GROUPS = CHUNK // LANES


def _emb_body(emb, new, idx, out, new_v, idx_v, idxc_v, rows_v, gsem):
    wid = lax.axis_index("s") * NC + lax.axis_index("c")
    b_per_w = idx.shape[0] // NW
    n_chunks = b_per_w // CHUNK
    base = wid * b_per_w
    pltpu.sync_copy(new, new_v)

    def do_chunk(g, carry):
        off = base + g * CHUNK
        pltpu.sync_copy(idx.at[pl.ds(off, CHUNK)], idx_v)

        def clamp(j, c):
            iv = idx_v[pl.ds(j * LANES, LANES)]
            idxc_v[pl.ds(j * LANES, LANES)] = jnp.minimum(iv, NUM_EMB - 1)
            return c

        lax.fori_loop(0, GROUPS, clamp, 0)
        pltpu.async_copy(emb.at[idxc_v], rows_v, gsem).wait()

        def fixup(j, c):
            iv = idx_v[pl.ds(j * LANES, LANES)]
            m = iv >= NUM_EMB
            gmax = jnp.max(iv)

            @pl.when(gmax >= NUM_EMB)
            def _():
                rn = jnp.clip(iv - NUM_EMB, 0, NUM_NEW - 1)
                rowpos = j * LANES + lax.iota(jnp.int32, LANES)
                for col in range(FEAT):
                    csplat = jnp.full((LANES,), col, jnp.int32)
                    vals = plsc.load_gather(new_v, [rn, csplat])
                    plsc.store_scatter(rows_v, [rowpos, csplat], vals, mask=m)

            return c

        lax.fori_loop(0, GROUPS, fixup, 0)
        pltpu.sync_copy(rows_v, out.at[pl.ds(off, CHUNK)])
        return carry

    lax.fori_loop(0, n_chunks, do_chunk, 0)


def _make_kernel(batch):
    return functools.partial(
        pl.kernel,
        out_type=jax.ShapeDtypeStruct((batch, FEAT), jnp.float32),
        mesh=plsc.VectorSubcoreMesh(
            core_axis_name="c", subcore_axis_name="s",
            num_cores=NC, num_subcores=NS,
        ),
        compiler_params=pltpu.CompilerParams(
            use_tc_tiling_on_sc=False, needs_layout_passes=False),
        scratch_types=[
            pltpu.VMEM((NUM_NEW, FEAT), jnp.float32),
            pltpu.VMEM((CHUNK,), jnp.int32),
            pltpu.VMEM((CHUNK,), jnp.int32),
            pltpu.VMEM((CHUNK, FEAT), jnp.float32),
            pltpu.SemaphoreType.DMA,
        ],
    )(_emb_body)


def kernel(x, embedding, new_embedding):
    idx = x.reshape(-1).astype(jnp.int32)
    out = _make_kernel(idx.shape[0])(embedding, new_embedding, idx)
    return out.reshape(x.shape + (FEAT,))


# speedup vs baseline: 5.3909x; 1.0145x over previous
import functools

import jax
import jax.numpy as jnp
from jax import lax
from jax.experimental import pallas as pl
from jax.experimental.pallas import tpu as pltpu
from jax.experimental.pallas import tpu_sc as plsc

NUM_EMB = 100000
NUM_NEW = 128
FEAT = 64
NC, NS, LANES = 2, 16, 16
NW = NC * NS
XROWS, XCOLS = 16384, 50
ROWS_PER_CHUNK = 16
CHUNK = ROWS_PER_CHUNK * XCOLS
GROUPS = CHUNK // LANES
ROWS_PER_W = XROWS // NW
N_CHUNKS = ROWS_PER_W // ROWS_PER_CHUNK


def _emb_body(emb, new, idx, out, new_v, idx2d_v, rg_v, cg_v, idxo_v, idxc_v,
              rows_v, gsem):
    wid = lax.axis_index("s") * NC + lax.axis_index("c")
    row0 = wid * ROWS_PER_W
    flat0 = row0 * XCOLS
    pltpu.sync_copy(new, new_v)

    def mk_tables(j, c):
        p = j * LANES + lax.iota(jnp.int32, LANES)
        rg_v[pl.ds(j * LANES, LANES)] = p // XCOLS
        cg_v[pl.ds(j * LANES, LANES)] = p % XCOLS
        return c

    lax.fori_loop(0, GROUPS, mk_tables, 0)

    def do_chunk(g, carry):
        pltpu.sync_copy(idx.at[pl.ds(row0 + g * ROWS_PER_CHUNK, ROWS_PER_CHUNK)],
                        idx2d_v)

        def flatten(j, c):
            s = pl.ds(j * LANES, LANES)
            iv = plsc.load_gather(idx2d_v, [rg_v[s], cg_v[s]])
            idxo_v[s] = iv
            idxc_v[s] = jnp.minimum(iv, NUM_EMB - 1)
            return c

        lax.fori_loop(0, GROUPS, flatten, 0)
        pltpu.async_copy(emb.at[idxc_v], rows_v, gsem).wait()

        def fixup(j, c):
            iv = idxo_v[pl.ds(j * LANES, LANES)]
            m = iv >= NUM_EMB
            gmax = jnp.max(iv)

            @pl.when(gmax >= NUM_EMB)
            def _():
                rn = jnp.clip(iv - NUM_EMB, 0, NUM_NEW - 1)
                rowpos = j * LANES + lax.iota(jnp.int32, LANES)
                for col in range(FEAT):
                    csplat = jnp.full((LANES,), col, jnp.int32)
                    vals = plsc.load_gather(new_v, [rn, csplat])
                    plsc.store_scatter(rows_v, [rowpos, csplat], vals, mask=m)

            return c

        lax.fori_loop(0, GROUPS, fixup, 0)
        pltpu.sync_copy(rows_v, out.at[pl.ds(flat0 + g * CHUNK, CHUNK)])
        return carry

    lax.fori_loop(0, N_CHUNKS, do_chunk, 0)


_emb_kernel = functools.partial(
    pl.kernel,
    out_type=jax.ShapeDtypeStruct((XROWS * XCOLS, FEAT), jnp.float32),
    mesh=plsc.VectorSubcoreMesh(
        core_axis_name="c", subcore_axis_name="s",
        num_cores=NC, num_subcores=NS,
    ),
    compiler_params=pltpu.CompilerParams(
        use_tc_tiling_on_sc=False, needs_layout_passes=False),
    scratch_types=[
        pltpu.VMEM((NUM_NEW, FEAT), jnp.float32),
        pltpu.VMEM((ROWS_PER_CHUNK, XCOLS), jnp.int32),
        pltpu.VMEM((CHUNK,), jnp.int32),
        pltpu.VMEM((CHUNK,), jnp.int32),
        pltpu.VMEM((CHUNK,), jnp.int32),
        pltpu.VMEM((CHUNK,), jnp.int32),
        pltpu.VMEM((CHUNK, FEAT), jnp.float32),
        pltpu.SemaphoreType.DMA,
    ],
)(_emb_body)


def kernel(x, embedding, new_embedding):
    out = _emb_kernel(embedding, new_embedding, x.astype(jnp.int32))
    return out.reshape(x.shape + (FEAT,))


# speedup vs baseline: 5.9101x; 1.0963x over previous
import functools

import jax
import jax.numpy as jnp
from jax import lax
from jax.experimental import pallas as pl
from jax.experimental.pallas import tpu as pltpu
from jax.experimental.pallas import tpu_sc as plsc

NUM_EMB = 100000
NUM_NEW = 128
FEAT = 64
NC, NS, LANES = 2, 16, 16
NW = NC * NS
XROWS, XCOLS = 16384, 50
ROWS_PER_CHUNK = 16
CHUNK = ROWS_PER_CHUNK * XCOLS
GROUPS = CHUNK // LANES
ROWS_PER_W = XROWS // NW
N_CHUNKS = ROWS_PER_W // ROWS_PER_CHUNK
NBUF = 2


def _emb_body(emb, new, idx, out, new_v, idx2d_v, rg_v, cg_v, idxo_v, idxc_v,
              rows_v, gsems, ssems):
    wid = lax.axis_index("s") * NC + lax.axis_index("c")
    row0 = wid * ROWS_PER_W
    flat0 = row0 * XCOLS
    pltpu.sync_copy(new, new_v)

    def mk_tables(j, c):
        p = j * LANES + lax.iota(jnp.int32, LANES)
        rg_v[pl.ds(j * LANES, LANES)] = p // XCOLS
        cg_v[pl.ds(j * LANES, LANES)] = p % XCOLS
        return c

    lax.fori_loop(0, GROUPS, mk_tables, 0)

    def out_slice(g):
        return out.at[pl.ds(flat0 + g * CHUNK, CHUNK)]

    def prep(g, b):
        pltpu.sync_copy(
            idx.at[pl.ds(row0 + g * ROWS_PER_CHUNK, ROWS_PER_CHUNK)],
            idx2d_v[b])

        def flatten(j, c):
            s = pl.ds(j * LANES, LANES)
            iv = plsc.load_gather(idx2d_v[b], [rg_v[s], cg_v[s]])
            idxo_v[b][s] = iv
            idxc_v[b][s] = jnp.minimum(iv, NUM_EMB - 1)
            return c

        lax.fori_loop(0, GROUPS, flatten, 0)
        pltpu.async_copy(emb.at[idxc_v[b]], rows_v[b], gsems[b])

    def fixup(b):
        def fix_group(j, c):
            iv = idxo_v[b][pl.ds(j * LANES, LANES)]
            m = iv >= NUM_EMB
            gmax = jnp.max(iv)

            @pl.when(gmax >= NUM_EMB)
            def _():
                rn = jnp.clip(iv - NUM_EMB, 0, NUM_NEW - 1)
                rowpos = j * LANES + lax.iota(jnp.int32, LANES)

                def fix_col(col, cc):
                    csplat = jnp.full((LANES,), col, jnp.int32)
                    vals = plsc.load_gather(new_v, [rn, csplat])
                    plsc.store_scatter(rows_v[b], [rowpos, csplat], vals,
                                       mask=m)
                    return cc

                lax.fori_loop(0, FEAT, fix_col, 0)

            return c

        lax.fori_loop(0, GROUPS, fix_group, 0)

    def finish(g, b):
        pltpu.make_async_copy(emb.at[idxc_v[b]], rows_v[b], gsems[b]).wait()
        fixup(b)
        pltpu.async_copy(rows_v[b], out_slice(g), ssems[b])

    def pair(t, carry):
        for b in range(NBUF):
            g = NBUF * t + b

            @pl.when(g >= NBUF)
            def _():
                pltpu.make_async_copy(rows_v[b], out_slice(g - NBUF),
                                      ssems[b]).wait()

            prep(g, b)

            @pl.when(g >= 1)
            def _():
                finish(g - 1, (b - 1) % NBUF)

        return carry

    lax.fori_loop(0, N_CHUNKS // NBUF, pair, 0)
    last = N_CHUNKS - 1
    lb = last % NBUF
    pltpu.make_async_copy(emb.at[idxc_v[lb]], rows_v[lb], gsems[lb]).wait()
    fixup(lb)
    pltpu.sync_copy(rows_v[lb], out_slice(last))
    pltpu.make_async_copy(rows_v[1 - lb], out_slice(last - 1),
                          ssems[1 - lb]).wait()


_emb_kernel = functools.partial(
    pl.kernel,
    out_type=jax.ShapeDtypeStruct((XROWS * XCOLS, FEAT), jnp.float32),
    mesh=plsc.VectorSubcoreMesh(
        core_axis_name="c", subcore_axis_name="s",
        num_cores=NC, num_subcores=NS,
    ),
    compiler_params=pltpu.CompilerParams(
        use_tc_tiling_on_sc=False, needs_layout_passes=False),
    scratch_types=[
        pltpu.VMEM((NUM_NEW, FEAT), jnp.float32),
        [pltpu.VMEM((ROWS_PER_CHUNK, XCOLS), jnp.int32)] * NBUF,
        pltpu.VMEM((CHUNK,), jnp.int32),
        pltpu.VMEM((CHUNK,), jnp.int32),
        [pltpu.VMEM((CHUNK,), jnp.int32)] * NBUF,
        [pltpu.VMEM((CHUNK,), jnp.int32)] * NBUF,
        [pltpu.VMEM((CHUNK, FEAT), jnp.float32)] * NBUF,
        [pltpu.SemaphoreType.DMA] * NBUF,
        [pltpu.SemaphoreType.DMA] * NBUF,
    ],
)(_emb_body)


def kernel(x, embedding, new_embedding):
    out = _emb_kernel(embedding, new_embedding, x.astype(jnp.int32))
    return out.reshape(x.shape + (FEAT,))


# speedup vs baseline: 5.9185x; 1.0014x over previous
import functools

import jax
import jax.numpy as jnp
from jax import lax
from jax.experimental import pallas as pl
from jax.experimental.pallas import tpu as pltpu
from jax.experimental.pallas import tpu_sc as plsc

NUM_EMB = 100000
NUM_NEW = 128
FEAT = 64
NC, NS, LANES = 2, 16, 16
NW = NC * NS
XROWS, XCOLS = 16384, 50
ROWS_PER_CHUNK = 16
CHUNK = ROWS_PER_CHUNK * XCOLS
GROUPS = CHUNK // LANES
ROWS_PER_W = XROWS // NW
N_CHUNKS = ROWS_PER_W // ROWS_PER_CHUNK
NBUF = 2


def _emb_body(emb, new, idx, out, new_v, idx2d_v, rg_v, cg_v, idxo_v, idxc_v,
              rows_v, gsems, ssems):
    wid = lax.axis_index("s") * NC + lax.axis_index("c")
    row0 = wid * ROWS_PER_W
    flat0 = row0 * XCOLS
    pltpu.sync_copy(new, new_v)

    def mk_tables(j, c):
        p = j * LANES + lax.iota(jnp.int32, LANES)
        rg_v[pl.ds(j * LANES, LANES)] = p // XCOLS
        cg_v[pl.ds(j * LANES, LANES)] = p % XCOLS
        return c

    lax.fori_loop(0, GROUPS, mk_tables, 0)

    def store_chunk(g, b, sem):
        r0 = row0 + g * ROWS_PER_CHUNK
        for k in range(ROWS_PER_CHUNK):
            pltpu.async_copy(rows_v[b].at[pl.ds(k * XCOLS, XCOLS), :],
                             out.at[r0 + k], sem)

    def drain_chunk(b, sem):
        for k in range(ROWS_PER_CHUNK):
            pltpu.make_async_copy(rows_v[b].at[pl.ds(k * XCOLS, XCOLS), :],
                                  out.at[row0 + k], sem).wait()

    def prep(g, b):
        pltpu.sync_copy(
            idx.at[pl.ds(row0 + g * ROWS_PER_CHUNK, ROWS_PER_CHUNK)],
            idx2d_v[b])

        def flatten(j, c):
            s = pl.ds(j * LANES, LANES)
            iv = plsc.load_gather(idx2d_v[b], [rg_v[s], cg_v[s]])
            idxo_v[b][s] = iv
            idxc_v[b][s] = jnp.minimum(iv, NUM_EMB - 1)
            return c

        lax.fori_loop(0, GROUPS, flatten, 0)
        pltpu.async_copy(emb.at[idxc_v[b]], rows_v[b], gsems[b])

    def fixup(b):
        def fix_group(j, c):
            iv = idxo_v[b][pl.ds(j * LANES, LANES)]
            m = iv >= NUM_EMB
            gmax = jnp.max(iv)

            @pl.when(gmax >= NUM_EMB)
            def _():
                rn = jnp.clip(iv - NUM_EMB, 0, NUM_NEW - 1)
                rowpos = j * LANES + lax.iota(jnp.int32, LANES)

                def fix_col(col, cc):
                    csplat = jnp.full((LANES,), col, jnp.int32)
                    vals = plsc.load_gather(new_v, [rn, csplat])
                    plsc.store_scatter(rows_v[b], [rowpos, csplat], vals,
                                       mask=m)
                    return cc

                lax.fori_loop(0, FEAT, fix_col, 0)

            return c

        lax.fori_loop(0, GROUPS, fix_group, 0)

    def finish(g, b):
        pltpu.make_async_copy(emb.at[idxc_v[b]], rows_v[b], gsems[b]).wait()
        fixup(b)
        store_chunk(g, b, ssems[b])

    def pair(t, carry):
        for b in range(NBUF):
            g = NBUF * t + b

            @pl.when(g >= NBUF)
            def _():
                drain_chunk(b, ssems[b])

            prep(g, b)

            @pl.when(g >= 1)
            def _():
                finish(g - 1, (b - 1) % NBUF)

        return carry

    lax.fori_loop(0, N_CHUNKS // NBUF, pair, 0)
    last = N_CHUNKS - 1
    lb = last % NBUF
    pltpu.make_async_copy(emb.at[idxc_v[lb]], rows_v[lb], gsems[lb]).wait()
    fixup(lb)
    store_chunk(last, lb, ssems[lb])
    drain_chunk(1 - lb, ssems[1 - lb])
    drain_chunk(lb, ssems[lb])


_emb_kernel = functools.partial(
    pl.kernel,
    out_type=jax.ShapeDtypeStruct((XROWS, XCOLS, FEAT), jnp.float32),
    mesh=plsc.VectorSubcoreMesh(
        core_axis_name="c", subcore_axis_name="s",
        num_cores=NC, num_subcores=NS,
    ),
    compiler_params=pltpu.CompilerParams(
        use_tc_tiling_on_sc=False, needs_layout_passes=False),
    scratch_types=[
        pltpu.VMEM((NUM_NEW, FEAT), jnp.float32),
        [pltpu.VMEM((ROWS_PER_CHUNK, XCOLS), jnp.int32)] * NBUF,
        pltpu.VMEM((CHUNK,), jnp.int32),
        pltpu.VMEM((CHUNK,), jnp.int32),
        [pltpu.VMEM((CHUNK,), jnp.int32)] * NBUF,
        [pltpu.VMEM((CHUNK,), jnp.int32)] * NBUF,
        [pltpu.VMEM((CHUNK, FEAT), jnp.float32)] * NBUF,
        [pltpu.SemaphoreType.DMA] * NBUF,
        [pltpu.SemaphoreType.DMA] * NBUF,
    ],
)(_emb_body)


def kernel(x, embedding, new_embedding):
    return _emb_kernel(embedding, new_embedding, x.astype(jnp.int32))


# speedup vs baseline: 6.9747x; 1.1785x over previous
import functools

import jax
import jax.numpy as jnp
from jax import lax
from jax.experimental import pallas as pl
from jax.experimental.pallas import tpu as pltpu
from jax.experimental.pallas import tpu_sc as plsc

NUM_EMB = 100000
NUM_NEW = 128
FEAT = 64
NC, NS, LANES = 2, 16, 16
NW = NC * NS
XROWS, XCOLS = 16384, 50
PADF = 128
HPAD = 56
ROWS_PER_CHUNK = 4
CHUNK = ROWS_PER_CHUNK * HPAD
GROUPS = CHUNK // LANES
ROWS_PER_W = XROWS // NW
N_CHUNKS = ROWS_PER_W // ROWS_PER_CHUNK
NBUF = 2


def _emb_body(emb, new, idx, out, new_v, idx2d_v, rg_v, cg_v, idxo_v, idxc_v,
              rows_v, gsems, ssems):
    wid = lax.axis_index("s") * NC + lax.axis_index("c")
    row0 = wid * ROWS_PER_W
    pltpu.sync_copy(new, new_v)

    def mk_tables(j, c):
        p = j * LANES + lax.iota(jnp.int32, LANES)
        rg_v[pl.ds(j * LANES, LANES)] = p // HPAD
        cg_v[pl.ds(j * LANES, LANES)] = jnp.minimum(p % HPAD, XCOLS - 1)
        return c

    lax.fori_loop(0, GROUPS, mk_tables, 0)

    def out_slice(g):
        return out.at[pl.ds((row0 + g * ROWS_PER_CHUNK) * HPAD, CHUNK), :]

    def prep(g, b):
        pltpu.sync_copy(
            idx.at[pl.ds(row0 + g * ROWS_PER_CHUNK, ROWS_PER_CHUNK)],
            idx2d_v[b])

        def flatten(j, c):
            s = pl.ds(j * LANES, LANES)
            iv = plsc.load_gather(idx2d_v[b], [rg_v[s], cg_v[s]])
            idxo_v[b][s] = iv
            idxc_v[b][s] = jnp.minimum(iv, NUM_EMB - 1)
            return c

        lax.fori_loop(0, GROUPS, flatten, 0)
        pltpu.async_copy(emb.at[idxc_v[b]], rows_v[b], gsems[b])

    def fixup(b):
        def fix_group(j, c):
            iv = idxo_v[b][pl.ds(j * LANES, LANES)]
            m = iv >= NUM_EMB
            gmax = jnp.max(iv)

            @pl.when(gmax >= NUM_EMB)
            def _():
                rn = jnp.clip(iv - NUM_EMB, 0, NUM_NEW - 1)
                rowpos = j * LANES + lax.iota(jnp.int32, LANES)

                def fix_col(col, cc):
                    csplat = jnp.full((LANES,), col, jnp.int32)
                    vals = plsc.load_gather(new_v, [rn, csplat])
                    plsc.store_scatter(rows_v[b], [rowpos, csplat], vals,
                                       mask=m)
                    return cc

                lax.fori_loop(0, FEAT, fix_col, 0)

            return c

        lax.fori_loop(0, GROUPS, fix_group, 0)

    def finish(g, b):
        pltpu.make_async_copy(emb.at[idxc_v[b]], rows_v[b], gsems[b]).wait()
        fixup(b)
        pltpu.async_copy(rows_v[b], out_slice(g), ssems[b])

    def pair(t, carry):
        for b in range(NBUF):
            g = NBUF * t + b

            @pl.when(g >= NBUF)
            def _():
                pltpu.make_async_copy(rows_v[b], out_slice(g - NBUF),
                                      ssems[b]).wait()

            prep(g, b)

            @pl.when(g >= 1)
            def _():
                finish(g - 1, (b - 1) % NBUF)

        return carry

    lax.fori_loop(0, N_CHUNKS // NBUF, pair, 0)
    last = N_CHUNKS - 1
    lb = last % NBUF
    pltpu.make_async_copy(emb.at[idxc_v[lb]], rows_v[lb], gsems[lb]).wait()
    fixup(lb)
    pltpu.sync_copy(rows_v[lb], out_slice(last))
    pltpu.make_async_copy(rows_v[1 - lb], out_slice(last - 1),
                          ssems[1 - lb]).wait()


_emb_kernel = functools.partial(
    pl.kernel,
    out_type=jax.ShapeDtypeStruct((XROWS * HPAD, PADF), jnp.float32),
    mesh=plsc.VectorSubcoreMesh(
        core_axis_name="c", subcore_axis_name="s",
        num_cores=NC, num_subcores=NS,
    ),
    compiler_params=pltpu.CompilerParams(
        use_tc_tiling_on_sc=True, needs_layout_passes=False),
    scratch_types=[
        pltpu.VMEM((NUM_NEW, PADF), jnp.float32),
        [pltpu.VMEM((ROWS_PER_CHUNK, PADF), jnp.int32)] * NBUF,
        pltpu.VMEM((CHUNK,), jnp.int32),
        pltpu.VMEM((CHUNK,), jnp.int32),
        [pltpu.VMEM((CHUNK,), jnp.int32)] * NBUF,
        [pltpu.VMEM((CHUNK,), jnp.int32)] * NBUF,
        [pltpu.VMEM((CHUNK, PADF), jnp.float32)] * NBUF,
        [pltpu.SemaphoreType.DMA] * NBUF,
        [pltpu.SemaphoreType.DMA] * NBUF,
    ],
)(_emb_body)


def kernel(x, embedding, new_embedding):
    embp = jnp.pad(embedding, ((0, 0), (0, PADF - FEAT)))
    newp = jnp.pad(new_embedding, ((0, 0), (0, PADF - FEAT)))
    xp = jnp.pad(x.astype(jnp.int32), ((0, 0), (0, PADF - XCOLS)))
    out = _emb_kernel(embp, newp, xp)
    return out.reshape(XROWS, HPAD, PADF)[:, :XCOLS, :FEAT]


# speedup vs baseline: 9.8419x; 1.4111x over previous
import functools

import jax
import jax.numpy as jnp
from jax import lax
from jax.experimental import pallas as pl
from jax.experimental.pallas import tpu as pltpu
from jax.experimental.pallas import tpu_sc as plsc

NUM_EMB = 100000
NUM_NEW = 128
FEAT = 64
NC, NS, LANES = 2, 16, 16
NW = NC * NS
XROWS, XCOLS = 16384, 50
PADF = 128
HPAD = 56
ROWS_PER_CHUNK = 8
CHUNK = ROWS_PER_CHUNK * XCOLS
GROUPS = CHUNK // LANES
ROWS_PER_W = XROWS // NW
N_CHUNKS = ROWS_PER_W // ROWS_PER_CHUNK
NBUF = 2


def _emb_body(emb, new, idx, out, new_v, idx2d_v, rg_v, cg_v, idxo_v, idxc_v,
              rows_v, gsems, ssems):
    wid = lax.axis_index("s") * NC + lax.axis_index("c")
    row0 = wid * ROWS_PER_W
    pltpu.sync_copy(new, new_v)

    def mk_tables(j, c):
        p = j * LANES + lax.iota(jnp.int32, LANES)
        rg_v[pl.ds(j * LANES, LANES)] = p // XCOLS
        cg_v[pl.ds(j * LANES, LANES)] = p % XCOLS
        return c

    lax.fori_loop(0, GROUPS, mk_tables, 0)

    def store_chunk(g, b, sem):
        r0 = row0 + g * ROWS_PER_CHUNK
        for k in range(ROWS_PER_CHUNK):
            pltpu.async_copy(
                rows_v[b].at[pl.ds(k * XCOLS, XCOLS), :],
                out.at[pl.ds((r0 + k) * HPAD, XCOLS), pl.ds(0, FEAT)], sem)

    def drain_chunk(b, sem):
        for k in range(ROWS_PER_CHUNK):
            pltpu.make_async_copy(
                rows_v[b].at[pl.ds(k * XCOLS, XCOLS), :],
                out.at[pl.ds(row0 * HPAD, XCOLS), pl.ds(0, FEAT)], sem).wait()

    def prep(g, b):
        pltpu.sync_copy(
            idx.at[pl.ds(row0 + g * ROWS_PER_CHUNK, ROWS_PER_CHUNK)],
            idx2d_v[b])

        def flatten(j, c):
            s = pl.ds(j * LANES, LANES)
            iv = plsc.load_gather(idx2d_v[b], [rg_v[s], cg_v[s]])
            idxo_v[b][s] = iv
            idxc_v[b][s] = jnp.minimum(iv, NUM_EMB - 1)
            return c

        lax.fori_loop(0, GROUPS, flatten, 0)
        pltpu.async_copy(emb.at[idxc_v[b]], rows_v[b], gsems[b])

    def fixup(b):
        def fix_group(j, c):
            iv = idxo_v[b][pl.ds(j * LANES, LANES)]
            m = iv >= NUM_EMB
            gmax = jnp.max(iv)

            @pl.when(gmax >= NUM_EMB)
            def _():
                rn = jnp.clip(iv - NUM_EMB, 0, NUM_NEW - 1)
                rowpos = j * LANES + lax.iota(jnp.int32, LANES)

                def fix_col(col, cc):
                    csplat = jnp.full((LANES,), col, jnp.int32)
                    vals = plsc.load_gather(new_v, [rn, csplat])
                    plsc.store_scatter(rows_v[b], [rowpos, csplat], vals,
                                       mask=m)
                    return cc

                lax.fori_loop(0, FEAT, fix_col, 0)

            return c

        lax.fori_loop(0, GROUPS, fix_group, 0)

    def finish(g, b):
        pltpu.make_async_copy(emb.at[idxc_v[b]], rows_v[b], gsems[b]).wait()
        fixup(b)
        store_chunk(g, b, ssems[b])

    def pair(t, carry):
        for b in range(NBUF):
            g = NBUF * t + b

            @pl.when(g >= NBUF)
            def _():
                drain_chunk(b, ssems[b])

            prep(g, b)

            @pl.when(g >= 1)
            def _():
                finish(g - 1, (b - 1) % NBUF)

        return carry

    lax.fori_loop(0, N_CHUNKS // NBUF, pair, 0)
    last = N_CHUNKS - 1
    lb = last % NBUF
    pltpu.make_async_copy(emb.at[idxc_v[lb]], rows_v[lb], gsems[lb]).wait()
    fixup(lb)
    store_chunk(last, lb, ssems[lb])
    drain_chunk(1 - lb, ssems[1 - lb])
    drain_chunk(lb, ssems[lb])


_emb_kernel = functools.partial(
    pl.kernel,
    out_type=jax.ShapeDtypeStruct((XROWS * HPAD, PADF), jnp.float32),
    mesh=plsc.VectorSubcoreMesh(
        core_axis_name="c", subcore_axis_name="s",
        num_cores=NC, num_subcores=NS,
    ),
    compiler_params=pltpu.CompilerParams(
        use_tc_tiling_on_sc=False, needs_layout_passes=False),
    scratch_types=[
        pltpu.VMEM((NUM_NEW, FEAT), jnp.float32),
        [pltpu.VMEM((ROWS_PER_CHUNK, XCOLS), jnp.int32)] * NBUF,
        pltpu.VMEM((CHUNK,), jnp.int32),
        pltpu.VMEM((CHUNK,), jnp.int32),
        [pltpu.VMEM((CHUNK,), jnp.int32)] * NBUF,
        [pltpu.VMEM((CHUNK,), jnp.int32)] * NBUF,
        [pltpu.VMEM((CHUNK, FEAT), jnp.float32)] * NBUF,
        [pltpu.SemaphoreType.DMA] * NBUF,
        [pltpu.SemaphoreType.DMA] * NBUF,
    ],
)(_emb_body)


def kernel(x, embedding, new_embedding):
    out = _emb_kernel(embedding, new_embedding, x.astype(jnp.int32))
    return out.reshape(XROWS, HPAD, PADF)[:, :XCOLS, :FEAT]


# speedup vs baseline: 10.4110x; 1.0578x over previous
import functools

import jax
import jax.numpy as jnp
from jax import lax
from jax.experimental import pallas as pl
from jax.experimental.pallas import tpu as pltpu
from jax.experimental.pallas import tpu_sc as plsc

NUM_EMB = 100000
NUM_NEW = 128
FEAT = 64
NC, NS, LANES = 2, 16, 16
NW = NC * NS
XROWS, XCOLS = 16384, 50
PADF = 128
HPAD = 56
ROWS_PER_CHUNK = 16
CHUNK = ROWS_PER_CHUNK * XCOLS
GROUPS = CHUNK // LANES
ROWS_PER_W = XROWS // NW
N_CHUNKS = ROWS_PER_W // ROWS_PER_CHUNK
NBUF = 2


def _emb_body(emb, new, idx, out, new_v, idx2d_v, rg_v, cg_v, idxo_v, idxc_v,
              rows_v, gsems, ssems, isems):
    wid = lax.axis_index("s") * NC + lax.axis_index("c")
    row0 = wid * ROWS_PER_W
    pltpu.sync_copy(new, new_v)

    def mk_tables(j, c):
        p = j * LANES + lax.iota(jnp.int32, LANES)
        rg_v[pl.ds(j * LANES, LANES)] = p // XCOLS
        cg_v[pl.ds(j * LANES, LANES)] = p % XCOLS
        return c

    lax.fori_loop(0, GROUPS, mk_tables, 0)
    for b0 in range(NBUF):
        pltpu.async_copy(idx.at[pl.ds(row0 + b0 * ROWS_PER_CHUNK,
                                      ROWS_PER_CHUNK)], idx2d_v[b0], isems[b0])

    def store_chunk(g, b, sem):
        r0 = row0 + g * ROWS_PER_CHUNK
        for k in range(ROWS_PER_CHUNK):
            pltpu.async_copy(
                rows_v[b].at[pl.ds(k * XCOLS, XCOLS), :],
                out.at[pl.ds((r0 + k) * HPAD, XCOLS), pl.ds(0, FEAT)], sem)

    def drain_chunk(b, sem):
        for k in range(ROWS_PER_CHUNK):
            pltpu.make_async_copy(
                rows_v[b].at[pl.ds(k * XCOLS, XCOLS), :],
                out.at[pl.ds(row0 * HPAD, XCOLS), pl.ds(0, FEAT)], sem).wait()

    def idx_slice(g):
        return idx.at[pl.ds(row0 + g * ROWS_PER_CHUNK, ROWS_PER_CHUNK)]

    def prep(g, b):
        pltpu.make_async_copy(idx_slice(g), idx2d_v[b], isems[b]).wait()

        def flatten(j, c):
            s = pl.ds(j * LANES, LANES)
            iv = plsc.load_gather(idx2d_v[b], [rg_v[s], cg_v[s]])
            idxo_v[b][s] = iv
            idxc_v[b][s] = jnp.minimum(iv, NUM_EMB - 1)
            return c

        lax.fori_loop(0, GROUPS, flatten, 0)

        @pl.when(g + NBUF < N_CHUNKS)
        def _():
            pltpu.async_copy(idx_slice(g + NBUF), idx2d_v[b], isems[b])

        pltpu.async_copy(emb.at[idxc_v[b]], rows_v[b], gsems[b])

    def fixup(b):
        def fix_group(j, c):
            iv = idxo_v[b][pl.ds(j * LANES, LANES)]
            m = iv >= NUM_EMB
            gmax = jnp.max(iv)

            @pl.when(gmax >= NUM_EMB)
            def _():
                rn = jnp.clip(iv - NUM_EMB, 0, NUM_NEW - 1)
                rowpos = j * LANES + lax.iota(jnp.int32, LANES)

                def fix_col(col, cc):
                    csplat = jnp.full((LANES,), col, jnp.int32)
                    vals = plsc.load_gather(new_v, [rn, csplat])
                    plsc.store_scatter(rows_v[b], [rowpos, csplat], vals,
                                       mask=m)
                    return cc

                lax.fori_loop(0, FEAT, fix_col, 0)

            return c

        lax.fori_loop(0, GROUPS, fix_group, 0)

    def finish(g, b):
        pltpu.make_async_copy(emb.at[idxc_v[b]], rows_v[b], gsems[b]).wait()
        fixup(b)
        store_chunk(g, b, ssems[b])

    def pair(t, carry):
        for b in range(NBUF):
            g = NBUF * t + b

            @pl.when(g >= NBUF)
            def _():
                drain_chunk(b, ssems[b])

            prep(g, b)

            @pl.when(g >= 1)
            def _():
                finish(g - 1, (b - 1) % NBUF)

        return carry

    lax.fori_loop(0, N_CHUNKS // NBUF, pair, 0)
    last = N_CHUNKS - 1
    lb = last % NBUF
    pltpu.make_async_copy(emb.at[idxc_v[lb]], rows_v[lb], gsems[lb]).wait()
    fixup(lb)
    store_chunk(last, lb, ssems[lb])
    drain_chunk(1 - lb, ssems[1 - lb])
    drain_chunk(lb, ssems[lb])


_emb_kernel = functools.partial(
    pl.kernel,
    out_type=jax.ShapeDtypeStruct((XROWS * HPAD, PADF), jnp.float32),
    mesh=plsc.VectorSubcoreMesh(
        core_axis_name="c", subcore_axis_name="s",
        num_cores=NC, num_subcores=NS,
    ),
    compiler_params=pltpu.CompilerParams(
        use_tc_tiling_on_sc=False, needs_layout_passes=False),
    scratch_types=[
        pltpu.VMEM((NUM_NEW, FEAT), jnp.float32),
        [pltpu.VMEM((ROWS_PER_CHUNK, XCOLS), jnp.int32)] * NBUF,
        pltpu.VMEM((CHUNK,), jnp.int32),
        pltpu.VMEM((CHUNK,), jnp.int32),
        [pltpu.VMEM((CHUNK,), jnp.int32)] * NBUF,
        [pltpu.VMEM((CHUNK,), jnp.int32)] * NBUF,
        [pltpu.VMEM((CHUNK, FEAT), jnp.float32)] * NBUF,
        [pltpu.SemaphoreType.DMA] * NBUF,
        [pltpu.SemaphoreType.DMA] * NBUF,
        [pltpu.SemaphoreType.DMA] * NBUF,
    ],
)(_emb_body)


def kernel(x, embedding, new_embedding):
    out = _emb_kernel(embedding, new_embedding, x.astype(jnp.int32))
    return out.reshape(XROWS, HPAD, PADF)[:, :XCOLS, :FEAT]


# speedup vs baseline: 10.4371x; 1.0025x over previous
import functools

import jax
import jax.numpy as jnp
from jax import lax
from jax.experimental import pallas as pl
from jax.experimental.pallas import tpu as pltpu
from jax.experimental.pallas import tpu_sc as plsc

NUM_EMB = 100000
NUM_NEW = 128
FEAT = 64
NC, NS, LANES = 2, 16, 16
NW = NC * NS
XROWS, XCOLS = 16384, 50
PADF = 128
HPAD = 56
ROWS_PER_CHUNK = 16
CHUNK = ROWS_PER_CHUNK * XCOLS
GROUPS = CHUNK // LANES
ROWS_PER_W = XROWS // NW
N_CHUNKS = ROWS_PER_W // ROWS_PER_CHUNK
NBUF = 2


def _emb_body(emb, new, idx, out, new_v, idx2d_v, rg_v, cg_v, idxo_v, idxc_v,
              rows_v, gsems, ssems, isems):
    wid = lax.axis_index("s") * NC + lax.axis_index("c")
    row0 = wid * ROWS_PER_W
    pltpu.sync_copy(new, new_v)

    def mk_tables(j, c):
        p = j * LANES + lax.iota(jnp.int32, LANES)
        rg_v[pl.ds(j * LANES, LANES)] = p // XCOLS
        cg_v[pl.ds(j * LANES, LANES)] = p % XCOLS
        return c

    lax.fori_loop(0, GROUPS, mk_tables, 0)
    for b0 in range(NBUF):
        pltpu.async_copy(idx.at[pl.ds(row0 + b0 * ROWS_PER_CHUNK,
                                      ROWS_PER_CHUNK)], idx2d_v[b0], isems[b0])

    def store_chunk(g, b, sem):
        r0 = row0 + g * ROWS_PER_CHUNK
        for k in range(ROWS_PER_CHUNK):
            pltpu.async_copy(
                rows_v[b].at[pl.ds(k * XCOLS, XCOLS), :],
                out.at[pl.ds((r0 + k) * HPAD, XCOLS), pl.ds(0, FEAT)], sem)

    def drain_chunk(b, sem):
        for k in range(ROWS_PER_CHUNK):
            pltpu.make_async_copy(
                rows_v[b].at[pl.ds(k * XCOLS, XCOLS), :],
                out.at[pl.ds(row0 * HPAD, XCOLS), pl.ds(0, FEAT)], sem).wait()

    def idx_slice(g):
        return idx.at[pl.ds(row0 + g * ROWS_PER_CHUNK, ROWS_PER_CHUNK)]

    def prep(g, b):
        pltpu.make_async_copy(idx_slice(g), idx2d_v[b], isems[b]).wait()

        def flatten(j, c):
            s = pl.ds(j * LANES, LANES)
            iv = plsc.load_gather(idx2d_v[b], [rg_v[s], cg_v[s]])
            idxo_v[b][s] = iv
            idxc_v[b][s] = jnp.minimum(iv, NUM_EMB - 1)
            return c

        lax.fori_loop(0, GROUPS, flatten, 0)

        @pl.when(g + NBUF < N_CHUNKS)
        def _():
            pltpu.async_copy(idx_slice(g + NBUF), idx2d_v[b], isems[b])

        @pl.when(g >= NBUF)
        def _():
            drain_chunk(b, ssems[b])

        pltpu.async_copy(emb.at[idxc_v[b]], rows_v[b], gsems[b])

    def fixup(b):
        def fix_group(j, c):
            iv = idxo_v[b][pl.ds(j * LANES, LANES)]
            m = iv >= NUM_EMB
            gmax = jnp.max(iv)

            @pl.when(gmax >= NUM_EMB)
            def _():
                rn = jnp.clip(iv - NUM_EMB, 0, NUM_NEW - 1)
                rowpos = j * LANES + lax.iota(jnp.int32, LANES)

                def fix_col(col, cc):
                    csplat = jnp.full((LANES,), col, jnp.int32)
                    vals = plsc.load_gather(new_v, [rn, csplat])
                    plsc.store_scatter(rows_v[b], [rowpos, csplat], vals,
                                       mask=m)
                    return cc

                lax.fori_loop(0, FEAT, fix_col, 0)

            return c

        lax.fori_loop(0, GROUPS, fix_group, 0)

    def finish(g, b):
        pltpu.make_async_copy(emb.at[idxc_v[b]], rows_v[b], gsems[b]).wait()
        fixup(b)
        store_chunk(g, b, ssems[b])

    def pair(t, carry):
        for b in range(NBUF):
            g = NBUF * t + b

            prep(g, b)

            @pl.when(g >= 1)
            def _():
                finish(g - 1, (b - 1) % NBUF)

        return carry

    lax.fori_loop(0, N_CHUNKS // NBUF, pair, 0)
    last = N_CHUNKS - 1
    lb = last % NBUF
    pltpu.make_async_copy(emb.at[idxc_v[lb]], rows_v[lb], gsems[lb]).wait()
    fixup(lb)
    store_chunk(last, lb, ssems[lb])
    drain_chunk(1 - lb, ssems[1 - lb])
    drain_chunk(lb, ssems[lb])


_emb_kernel = functools.partial(
    pl.kernel,
    out_type=jax.ShapeDtypeStruct((XROWS * HPAD, PADF), jnp.float32),
    mesh=plsc.VectorSubcoreMesh(
        core_axis_name="c", subcore_axis_name="s",
        num_cores=NC, num_subcores=NS,
    ),
    compiler_params=pltpu.CompilerParams(
        use_tc_tiling_on_sc=False, needs_layout_passes=False),
    scratch_types=[
        pltpu.VMEM((NUM_NEW, FEAT), jnp.float32),
        [pltpu.VMEM((ROWS_PER_CHUNK, XCOLS), jnp.int32)] * NBUF,
        pltpu.VMEM((CHUNK,), jnp.int32),
        pltpu.VMEM((CHUNK,), jnp.int32),
        [pltpu.VMEM((CHUNK,), jnp.int32)] * NBUF,
        [pltpu.VMEM((CHUNK,), jnp.int32)] * NBUF,
        [pltpu.VMEM((CHUNK, FEAT), jnp.float32)] * NBUF,
        [pltpu.SemaphoreType.DMA] * NBUF,
        [pltpu.SemaphoreType.DMA] * NBUF,
        [pltpu.SemaphoreType.DMA] * NBUF,
    ],
)(_emb_body)


def kernel(x, embedding, new_embedding):
    out = _emb_kernel(embedding, new_embedding, x.astype(jnp.int32))
    return out.reshape(XROWS, HPAD, PADF)[:, :XCOLS, :FEAT]


# speedup vs baseline: 10.5717x; 1.0129x over previous
import functools

import jax
import jax.numpy as jnp
from jax import lax
from jax.experimental import pallas as pl
from jax.experimental.pallas import tpu as pltpu
from jax.experimental.pallas import tpu_sc as plsc

NUM_EMB = 100000
NUM_NEW = 128
FEAT = 64
NC, NS, LANES = 2, 16, 16
NW = NC * NS
XROWS, XCOLS = 16384, 50
PADF = 128
HPAD = 56
ROWS_PER_CHUNK = 8
CHUNK = ROWS_PER_CHUNK * XCOLS
GROUPS = CHUNK // LANES
ROWS_PER_W = XROWS // NW
N_CHUNKS = ROWS_PER_W // ROWS_PER_CHUNK
NBUF = 4


def _emb_body(emb, new, idx, out, new_v, idx2d_v, rg_v, cg_v, idxo_v, idxc_v,
              rows_v, gsems, ssems, isems):
    wid = lax.axis_index("s") * NC + lax.axis_index("c")
    row0 = wid * ROWS_PER_W
    pltpu.sync_copy(new, new_v)

    def mk_tables(j, c):
        p = j * LANES + lax.iota(jnp.int32, LANES)
        rg_v[pl.ds(j * LANES, LANES)] = p // XCOLS
        cg_v[pl.ds(j * LANES, LANES)] = p % XCOLS
        return c

    lax.fori_loop(0, GROUPS, mk_tables, 0)
    for b0 in range(NBUF):
        pltpu.async_copy(idx.at[pl.ds(row0 + b0 * ROWS_PER_CHUNK,
                                      ROWS_PER_CHUNK)], idx2d_v[b0], isems[b0])

    def store_chunk(g, b, sem):
        r0 = row0 + g * ROWS_PER_CHUNK
        for k in range(ROWS_PER_CHUNK):
            pltpu.async_copy(
                rows_v[b].at[pl.ds(k * XCOLS, XCOLS), :],
                out.at[pl.ds((r0 + k) * HPAD, XCOLS), pl.ds(0, FEAT)], sem)

    def drain_chunk(b, sem):
        for k in range(ROWS_PER_CHUNK):
            pltpu.make_async_copy(
                rows_v[b].at[pl.ds(k * XCOLS, XCOLS), :],
                out.at[pl.ds(row0 * HPAD, XCOLS), pl.ds(0, FEAT)], sem).wait()

    def idx_slice(g):
        return idx.at[pl.ds(row0 + g * ROWS_PER_CHUNK, ROWS_PER_CHUNK)]

    def prep(g, b):
        pltpu.make_async_copy(idx_slice(g), idx2d_v[b], isems[b]).wait()

        def flatten(j, c):
            s = pl.ds(j * LANES, LANES)
            iv = plsc.load_gather(idx2d_v[b], [rg_v[s], cg_v[s]])
            idxo_v[b][s] = iv
            idxc_v[b][s] = jnp.minimum(iv, NUM_EMB - 1)
            return c

        lax.fori_loop(0, GROUPS, flatten, 0)

        @pl.when(g + NBUF < N_CHUNKS)
        def _():
            pltpu.async_copy(idx_slice(g + NBUF), idx2d_v[b], isems[b])

        @pl.when(g >= NBUF)
        def _():
            drain_chunk(b, ssems[b])

        pltpu.async_copy(emb.at[idxc_v[b]], rows_v[b], gsems[b])

    def fixup(b):
        def fix_group(j, c):
            iv = idxo_v[b][pl.ds(j * LANES, LANES)]
            m = iv >= NUM_EMB
            gmax = jnp.max(iv)

            @pl.when(gmax >= NUM_EMB)
            def _():
                rn = jnp.clip(iv - NUM_EMB, 0, NUM_NEW - 1)
                rowpos = j * LANES + lax.iota(jnp.int32, LANES)

                def fix_col(col, cc):
                    csplat = jnp.full((LANES,), col, jnp.int32)
                    vals = plsc.load_gather(new_v, [rn, csplat])
                    plsc.store_scatter(rows_v[b], [rowpos, csplat], vals,
                                       mask=m)
                    return cc

                lax.fori_loop(0, FEAT, fix_col, 0)

            return c

        lax.fori_loop(0, GROUPS, fix_group, 0)

    def finish(g, b):
        pltpu.make_async_copy(emb.at[idxc_v[b]], rows_v[b], gsems[b]).wait()
        fixup(b)
        store_chunk(g, b, ssems[b])

    def pair(t, carry):
        for b in range(NBUF):
            g = NBUF * t + b

            prep(g, b)

            @pl.when(g >= 2)
            def _():
                finish(g - 2, (b - 2) % NBUF)

        return carry

    lax.fori_loop(0, N_CHUNKS // NBUF, pair, 0)
    last = N_CHUNKS - 1
    finish(last - 1, (last - 1) % NBUF)
    finish(last, last % NBUF)
    for i in range(NBUF):
        drain_chunk(i, ssems[i])


_emb_kernel = functools.partial(
    pl.kernel,
    out_type=jax.ShapeDtypeStruct((XROWS * HPAD, PADF), jnp.float32),
    mesh=plsc.VectorSubcoreMesh(
        core_axis_name="c", subcore_axis_name="s",
        num_cores=NC, num_subcores=NS,
    ),
    compiler_params=pltpu.CompilerParams(
        use_tc_tiling_on_sc=False, needs_layout_passes=False),
    scratch_types=[
        pltpu.VMEM((NUM_NEW, FEAT), jnp.float32),
        [pltpu.VMEM((ROWS_PER_CHUNK, XCOLS), jnp.int32)] * NBUF,
        pltpu.VMEM((CHUNK,), jnp.int32),
        pltpu.VMEM((CHUNK,), jnp.int32),
        [pltpu.VMEM((CHUNK,), jnp.int32)] * NBUF,
        [pltpu.VMEM((CHUNK,), jnp.int32)] * NBUF,
        [pltpu.VMEM((CHUNK, FEAT), jnp.float32)] * NBUF,
        [pltpu.SemaphoreType.DMA] * NBUF,
        [pltpu.SemaphoreType.DMA] * NBUF,
        [pltpu.SemaphoreType.DMA] * NBUF,
    ],
)(_emb_body)


def kernel(x, embedding, new_embedding):
    out = _emb_kernel(embedding, new_embedding, x.astype(jnp.int32))
    return out.reshape(XROWS, HPAD, PADF)[:, :XCOLS, :FEAT]
